# Initial kernel scaffold; baseline (speedup 1.0000x reference)
import jax
import jax.numpy as jnp
from jax.experimental import pallas as pl


def kernel(idx, table):
    raise NotImplementedError("write your pallas kernel here")


# speedup vs baseline: 1.0270x; 1.0270x over previous
import functools

import jax
import jax.numpy as jnp
from jax import lax
from jax.experimental import pallas as pl
from jax.experimental.pallas import tpu as pltpu
from jax.experimental.pallas import tpu_sc as plsc

NUM_CORES = 2
NUM_SUBCORES = 16
NW = NUM_CORES * NUM_SUBCORES
CHUNK = 64


@functools.lru_cache(maxsize=None)
def _build_gather(n_rows: int, depth: int):
    n_per_w = n_rows // NW
    n_chunks = n_per_w // CHUNK
    assert n_per_w * NW == n_rows and n_chunks * CHUNK == n_per_w
    assert n_chunks % 2 == 1

    mesh = plsc.VectorSubcoreMesh(
        core_axis_name="c", subcore_axis_name="s",
        num_cores=NUM_CORES, num_subcores=NUM_SUBCORES)

    @functools.partial(
        pl.kernel,
        mesh=mesh,
        compiler_params=pltpu.CompilerParams(use_tc_tiling_on_sc=False),
        out_type=jax.ShapeDtypeStruct((n_rows, depth), jnp.float32),
        scratch_types=[
            pltpu.VMEM((n_per_w,), jnp.int32),
            pltpu.VMEM((CHUNK, depth), jnp.float32),
            pltpu.VMEM((CHUNK, depth), jnp.float32),
            pltpu.SemaphoreType.DMA,
            pltpu.SemaphoreType.DMA,
        ],
    )
    def gather_kernel(idx_hbm, table_hbm, out_hbm, idx_v, buf0, buf1, sem0, sem1):
        wid = lax.axis_index("s") * NUM_CORES + lax.axis_index("c")
        base = wid * n_per_w
        pltpu.sync_copy(idx_hbm.at[pl.ds(base, n_per_w)], idx_v)

        def start(j, buf, sem):
            pltpu.async_copy(
                table_hbm.at[idx_v.at[pl.ds(j * CHUNK, CHUNK)]], buf, sem)

        def wait(buf, sem):
            pltpu.make_async_copy(table_hbm.at[pl.ds(0, CHUNK)], buf, sem).wait()

        def put(j, buf):
            pltpu.sync_copy(buf, out_hbm.at[pl.ds(base + j * CHUNK, CHUNK)])

        start(0, buf0, sem0)

        @pl.loop(0, n_chunks - 1, step=2)
        def _pair(j):
            wait(buf0, sem0)
            start(j + 1, buf1, sem1)
            put(j, buf0)
            wait(buf1, sem1)
            start(j + 2, buf0, sem0)
            put(j + 1, buf1)

        wait(buf0, sem0)
        put(n_chunks - 1, buf0)

    return gather_kernel


def kernel(idx, table):
    b, t = idx.shape
    _, depth = table.shape
    n_rows = b * t
    idx_flat = idx.reshape(n_rows).astype(jnp.int32)
    out = _build_gather(n_rows, depth)(idx_flat, table)
    return out.reshape(b, t, depth)


# speedup vs baseline: 1.0304x; 1.0033x over previous
import functools

import jax
import jax.numpy as jnp
from jax import lax
from jax.experimental import pallas as pl
from jax.experimental.pallas import tpu as pltpu
from jax.experimental.pallas import tpu_sc as plsc

NUM_CORES = 2
NUM_SUBCORES = 16
NW = NUM_CORES * NUM_SUBCORES


@functools.lru_cache(maxsize=None)
def _build_gather(b: int, t: int, depth: int):
    b_per_w = b // NW
    assert b_per_w * NW == b and b_per_w % 2 == 0

    mesh = plsc.VectorSubcoreMesh(
        core_axis_name="c", subcore_axis_name="s",
        num_cores=NUM_CORES, num_subcores=NUM_SUBCORES)

    @functools.partial(
        pl.kernel,
        mesh=mesh,
        compiler_params=pltpu.CompilerParams(use_tc_tiling_on_sc=False),
        out_type=jax.ShapeDtypeStruct((b, t, depth), jnp.float32),
        scratch_types=[
            pltpu.VMEM((b_per_w, t), jnp.int32),
            pltpu.VMEM((t, depth), jnp.float32),
            pltpu.VMEM((t, depth), jnp.float32),
            pltpu.SemaphoreType.DMA,
            pltpu.SemaphoreType.DMA,
        ],
    )
    def gather_kernel(idx_hbm, table_hbm, out_hbm, idx_v, buf0, buf1, sem0, sem1):
        wid = lax.axis_index("s") * NUM_CORES + lax.axis_index("c")
        base = wid * b_per_w
        pltpu.sync_copy(idx_hbm.at[pl.ds(base, b_per_w)], idx_v)

        def start(j, buf, sem):
            pltpu.async_copy(table_hbm.at[idx_v.at[j]], buf, sem)

        def wait(buf, sem):
            pltpu.make_async_copy(table_hbm.at[pl.ds(0, t)], buf, sem).wait()

        def put(j, buf):
            pltpu.sync_copy(buf, out_hbm.at[base + j])

        start(0, buf0, sem0)

        @pl.loop(0, b_per_w, step=2)
        def _pair(j):
            wait(buf0, sem0)
            start(j + 1, buf1, sem1)
            put(j, buf0)
            wait(buf1, sem1)

            @pl.when(j + 2 < b_per_w)
            def _():
                start(j + 2, buf0, sem0)

            put(j + 1, buf1)

    return gather_kernel


def kernel(idx, table):
    b, t = idx.shape
    _, depth = table.shape
    return _build_gather(b, t, depth)(idx.astype(jnp.int32), table)


# speedup vs baseline: 1.4356x; 1.3933x over previous
import functools

import jax
import jax.numpy as jnp
from jax import lax
from jax.experimental import pallas as pl
from jax.experimental.pallas import tpu as pltpu
from jax.experimental.pallas import tpu_sc as plsc

NUM_CORES = 2
NUM_SUBCORES = 16
NW = NUM_CORES * NUM_SUBCORES
LANES = 16


@functools.lru_cache(maxsize=None)
def _build_gather(b: int, t: int, depth: int):
    assert b % 128 == 0 and depth % 8 == 0
    n_tiles = depth // 8
    n_bblk = b // 128
    chunk = 8 * 128 * n_bblk
    tiles_base = n_tiles // NW
    tiles_rem = n_tiles % NW
    max_tiles = tiles_base + (1 if tiles_rem else 0)
    assert t % 2 == 0

    mesh = plsc.VectorSubcoreMesh(
        core_axis_name="c", subcore_axis_name="s",
        num_cores=NUM_CORES, num_subcores=NUM_SUBCORES)

    @functools.partial(
        pl.kernel,
        mesh=mesh,
        compiler_params=pltpu.CompilerParams(
            use_tc_tiling_on_sc=False, needs_layout_passes=False),
        out_type=jax.ShapeDtypeStruct((t, n_tiles, chunk), jnp.float32),
        scratch_types=[
            pltpu.VMEM((max_tiles * 8, depth), jnp.float32),
            pltpu.VMEM((b,), jnp.int32),
            pltpu.VMEM((b,), jnp.int32),
            [pltpu.VMEM((chunk,), jnp.float32) for _ in range(max_tiles)],
            pltpu.SemaphoreType.DMA,
            pltpu.SemaphoreType.DMA,
            [pltpu.SemaphoreType.DMA for _ in range(max_tiles)],
        ],
    )
    def gather_kernel(idxT_hbm, tableT_hbm, out_hbm, slab, idx0, idx1,
                      obufs, isem0, isem1, osems):
        w = lax.axis_index("s") * NUM_CORES + lax.axis_index("c")
        lo = w * tiles_base + jnp.minimum(w, tiles_rem)
        nt = jnp.where(w < tiles_rem, tiles_base + 1, tiles_base)

        for k in range(max_tiles):
            @pl.when(k < nt)
            def _(k=k):
                pltpu.sync_copy(tableT_hbm.at[pl.ds((lo + k) * 8, 8)],
                                slab.at[pl.ds(k * 8, 8)])

        def compute_t(tt, cur):
            for k in range(max_tiles):
                buf, osem = obufs[k], osems[k]

                @pl.when(k < nt)
                def _(k=k, buf=buf, osem=osem):
                    @pl.when(tt > 0)
                    def _():
                        pltpu.make_async_copy(out_hbm.at[0, 0], buf, osem).wait()

                    @pl.loop(0, n_bblk)
                    def _bb(bb):
                        ivs = [cur[pl.ds(bb * 128 + j * LANES, LANES)]
                               for j in range(128 // LANES)]
                        for di in range(8):
                            row = slab.at[k * 8 + di]
                            for j in range(128 // LANES):
                                vals = plsc.load_gather(row, [ivs[j]])
                                buf[pl.ds(bb * 1024 + di * 128 + j * LANES,
                                          LANES)] = vals

                    pltpu.async_copy(buf, out_hbm.at[tt, lo + k], osem)

        pltpu.async_copy(idxT_hbm.at[0], idx0, isem0)

        @pl.loop(0, t, step=2)
        def _tpair(tt):
            pltpu.make_async_copy(idxT_hbm.at[0], idx0, isem0).wait()

            @pl.when(tt + 1 < t)
            def _():
                pltpu.async_copy(idxT_hbm.at[tt + 1], idx1, isem1)

            compute_t(tt, idx0)
            pltpu.make_async_copy(idxT_hbm.at[0], idx1, isem1).wait()

            @pl.when(tt + 2 < t)
            def _():
                pltpu.async_copy(idxT_hbm.at[tt + 2], idx0, isem0)

            compute_t(tt + 1, idx1)

        for k in range(max_tiles):
            @pl.when(k < nt)
            def _(k=k):
                pltpu.make_async_copy(out_hbm.at[0, 0], obufs[k], osems[k]).wait()

    return gather_kernel


def kernel(idx, table):
    b, t = idx.shape
    _, depth = table.shape
    idx_t = idx.T.astype(jnp.int32)
    table_t = table.T
    x = _build_gather(b, t, depth)(idx_t, table_t)
    return (x.reshape(t, depth // 8, b // 128, 8, 128)
            .transpose(2, 4, 0, 1, 3).reshape(b, t, depth))


# speedup vs baseline: 3.9708x; 2.7660x over previous
import functools

import jax
import jax.numpy as jnp
from jax import lax
from jax.experimental import pallas as pl
from jax.experimental.pallas import tpu as pltpu
from jax.experimental.pallas import tpu_sc as plsc

NUM_CORES = 2
NUM_SUBCORES = 16
NW = NUM_CORES * NUM_SUBCORES
LANES = 16


@functools.lru_cache(maxsize=None)
def _build_gather(b: int, t: int, depth: int):
    assert b % 128 == 0 and depth % 8 == 0
    n_tiles = depth // 8
    n_bblk = b // 128
    chunk = 8 * 128 * n_bblk
    tiles_base = n_tiles // NW
    tiles_rem = n_tiles % NW
    max_tiles = tiles_base + (1 if tiles_rem else 0)
    assert t % 2 == 0

    mesh = plsc.VectorSubcoreMesh(
        core_axis_name="c", subcore_axis_name="s",
        num_cores=NUM_CORES, num_subcores=NUM_SUBCORES)

    @functools.partial(
        pl.kernel,
        mesh=mesh,
        compiler_params=pltpu.CompilerParams(
            use_tc_tiling_on_sc=False, needs_layout_passes=False),
        out_type=jax.ShapeDtypeStruct((t, n_tiles, chunk), jnp.float32),
        scratch_types=[
            pltpu.VMEM((max_tiles * 8, depth), jnp.float32),
            pltpu.VMEM((b,), jnp.int32),
            pltpu.VMEM((b,), jnp.int32),
            [pltpu.VMEM((chunk,), jnp.float32) for _ in range(max_tiles)],
            pltpu.SemaphoreType.DMA,
            pltpu.SemaphoreType.DMA,
            [pltpu.SemaphoreType.DMA for _ in range(max_tiles)],
        ],
    )
    def gather_kernel(idxT_hbm, tableT_hbm, out_hbm, slab, idx0, idx1,
                      obufs, isem0, isem1, osems):
        w = lax.axis_index("s") * NUM_CORES + lax.axis_index("c")
        lo = w * tiles_base + jnp.minimum(w, tiles_rem)
        nt = jnp.where(w < tiles_rem, tiles_base + 1, tiles_base)

        for k in range(max_tiles):
            @pl.when(k < nt)
            def _(k=k):
                pltpu.sync_copy(tableT_hbm.at[pl.ds((lo + k) * 8, 8)],
                                slab.at[pl.ds(k * 8, 8)])

        def compute_t(tt, cur):
            for k in range(max_tiles):
                buf, osem = obufs[k], osems[k]

                @pl.when(k < nt)
                def _(k=k, buf=buf, osem=osem):
                    @pl.when(tt > 0)
                    def _():
                        pltpu.make_async_copy(out_hbm.at[0, 0], buf, osem).wait()

                    @pl.loop(0, n_bblk)
                    def _bb(bb):
                        ivs = [cur[pl.ds(bb * 128 + j * LANES, LANES)]
                               for j in range(128 // LANES)]
                        for di in range(8):
                            row = slab.at[k * 8 + di]
                            vals = [plsc.load_gather(row, [iv]) for iv in ivs]
                            for j, v in enumerate(vals):
                                buf[pl.ds(bb * 1024 + di * 128 + j * LANES,
                                          LANES)] = v

                    pltpu.async_copy(buf, out_hbm.at[tt, lo + k], osem)

        pltpu.async_copy(idxT_hbm.at[0], idx0, isem0)

        @pl.loop(0, t, step=2)
        def _tpair(tt):
            pltpu.make_async_copy(idxT_hbm.at[0], idx0, isem0).wait()

            @pl.when(tt + 1 < t)
            def _():
                pltpu.async_copy(idxT_hbm.at[tt + 1], idx1, isem1)

            compute_t(tt, idx0)
            pltpu.make_async_copy(idxT_hbm.at[0], idx1, isem1).wait()

            @pl.when(tt + 2 < t)
            def _():
                pltpu.async_copy(idxT_hbm.at[tt + 2], idx0, isem0)

            compute_t(tt + 1, idx1)

        for k in range(max_tiles):
            @pl.when(k < nt)
            def _(k=k):
                pltpu.make_async_copy(out_hbm.at[0, 0], obufs[k], osems[k]).wait()

    return gather_kernel


def kernel(idx, table):
    b, t = idx.shape
    _, depth = table.shape
    idx_t = idx.T.astype(jnp.int32)
    table_t = table.T
    x = _build_gather(b, t, depth)(idx_t, table_t)
    return (x.reshape(t, depth // 8, b // 128, 8, 128)
            .transpose(2, 4, 0, 1, 3).reshape(b, t, depth))


# speedup vs baseline: 4.2279x; 1.0647x over previous
import functools

import jax
import jax.numpy as jnp
from jax import lax
from jax.experimental import pallas as pl
from jax.experimental.pallas import tpu as pltpu
from jax.experimental.pallas import tpu_sc as plsc

NUM_CORES = 2
NUM_SUBCORES = 16
NW = NUM_CORES * NUM_SUBCORES
LANES = 16


@functools.lru_cache(maxsize=None)
def _build_gather(b: int, t: int, depth: int):
    assert b % 128 == 0 and depth % 8 == 0
    n_tiles = depth // 8
    n_bblk = b // 128
    chunk = 8 * 128 * n_bblk
    tiles_base = n_tiles // NW
    tiles_rem = n_tiles % NW
    max_tiles = tiles_base + (1 if tiles_rem else 0)
    assert t % 2 == 0

    mesh = plsc.VectorSubcoreMesh(
        core_axis_name="c", subcore_axis_name="s",
        num_cores=NUM_CORES, num_subcores=NUM_SUBCORES)

    @functools.partial(
        pl.kernel,
        mesh=mesh,
        compiler_params=pltpu.CompilerParams(
            use_tc_tiling_on_sc=False, needs_layout_passes=False),
        out_type=jax.ShapeDtypeStruct((t, n_tiles, chunk), jnp.float32),
        scratch_types=[
            pltpu.VMEM((max_tiles * 8, depth), jnp.float32),
            pltpu.VMEM((b,), jnp.int32),
            pltpu.VMEM((b,), jnp.int32),
            [pltpu.VMEM((chunk,), jnp.float32) for _ in range(max_tiles)],
            pltpu.SemaphoreType.DMA,
            pltpu.SemaphoreType.DMA,
            [pltpu.SemaphoreType.DMA for _ in range(max_tiles)],
        ],
    )
    def gather_kernel(idxT_hbm, tableT_hbm, out_hbm, slab, idx0, idx1,
                      obufs, isem0, isem1, osems):
        w = lax.axis_index("s") * NUM_CORES + lax.axis_index("c")
        lo = w * tiles_base + jnp.minimum(w, tiles_rem)
        nt = jnp.where(w < tiles_rem, tiles_base + 1, tiles_base)

        for k in range(max_tiles):
            @pl.when(k < nt)
            def _(k=k):
                pltpu.sync_copy(tableT_hbm.at[pl.ds((lo + k) * 8, 8)],
                                slab.at[pl.ds(k * 8, 8)])

        def compute_t(tt, cur):
            for k in range(max_tiles):
                buf, osem = obufs[k], osems[k]

                @pl.when(k < nt)
                def _(k=k, buf=buf, osem=osem):
                    @pl.when(tt > 0)
                    def _():
                        pltpu.make_async_copy(out_hbm.at[0, 0], buf, osem).wait()

                    @pl.loop(0, n_bblk)
                    def _bb(bb):
                        ivs = [cur[pl.ds(bb * 128 + j * LANES, LANES)]
                               for j in range(128 // LANES)]

                        def store(di, j, v):
                            buf[pl.ds(bb * 1024 + di * 128 + j * LANES,
                                      LANES)] = v

                        prev = [plsc.load_gather(slab.at[k * 8], [iv])
                                for iv in ivs]
                        for di in range(1, 8):
                            row = slab.at[k * 8 + di]
                            curr = []
                            for j, iv in enumerate(ivs):
                                store(di - 1, j, prev[j])
                                curr.append(plsc.load_gather(row, [iv]))
                            prev = curr
                        for j in range(len(ivs)):
                            store(7, j, prev[j])

                    pltpu.async_copy(buf, out_hbm.at[tt, lo + k], osem)

        pltpu.async_copy(idxT_hbm.at[0], idx0, isem0)

        @pl.loop(0, t, step=2)
        def _tpair(tt):
            pltpu.make_async_copy(idxT_hbm.at[0], idx0, isem0).wait()

            @pl.when(tt + 1 < t)
            def _():
                pltpu.async_copy(idxT_hbm.at[tt + 1], idx1, isem1)

            compute_t(tt, idx0)
            pltpu.make_async_copy(idxT_hbm.at[0], idx1, isem1).wait()

            @pl.when(tt + 2 < t)
            def _():
                pltpu.async_copy(idxT_hbm.at[tt + 2], idx0, isem0)

            compute_t(tt + 1, idx1)

        for k in range(max_tiles):
            @pl.when(k < nt)
            def _(k=k):
                pltpu.make_async_copy(out_hbm.at[0, 0], obufs[k], osems[k]).wait()

    return gather_kernel


def kernel(idx, table):
    b, t = idx.shape
    _, depth = table.shape
    idx_t = idx.T.astype(jnp.int32)
    table_t = table.T
    x = _build_gather(b, t, depth)(idx_t, table_t)
    return (x.reshape(t, depth // 8, b // 128, 8, 128)
            .transpose(2, 4, 0, 1, 3).reshape(b, t, depth))
